# initial kernel scaffold (unmeasured)
import jax
import jax.numpy as jnp
from jax import lax
from jax.experimental import pallas as pl
from jax.experimental.pallas import tpu as pltpu

N_DEV = 4
M_LOC = 1024
K = 4096
N_GLOB = 8192
N_LOC = N_GLOB // N_DEV
CB = 1024
NBLK = N_LOC // CB

GELU_C = 0.7978845608028654


def _gelu(y):
    return 0.5 * y * (1.0 + jnp.tanh(GELU_C * (y + 0.044715 * y * y * y)))


def kernel(x, w_mat):
    def body(x_ref, w_hbm, out_hbm, w_buf, send_buf, recv_buf, out_stage,
             w_sems, send_sems, recv_sems, out_sems):
        my = lax.axis_index("i")

        barrier = pltpu.get_barrier_semaphore()
        for s in range(1, N_DEV):
            pl.semaphore_signal(
                barrier, inc=1,
                device_id=((my + s) % N_DEV,),
                device_id_type=pl.DeviceIdType.MESH,
            )
        pl.semaphore_wait(barrier, N_DEV - 1)

        def w_load(t, j, slot):
            return pltpu.make_async_copy(
                w_hbm.at[:, pl.ds((t * NBLK + j) * CB, CB)],
                w_buf.at[slot],
                w_sems.at[slot],
            )

        order = [(s, j) for s in (1, 2, 3, 0) for j in range(NBLK)]

        loads = {}
        t0 = (my + order[0][0]) % N_DEV
        loads[0] = w_load(t0, order[0][1], 0)
        loads[0].start()

        out_inflight = [None, None]
        send_rdmas = []
        oslot = 0

        def store_out(value_f32, row_start, col_block):
            nonlocal oslot
            if out_inflight[oslot] is not None:
                out_inflight[oslot].wait()
            out_stage[oslot] = value_f32
            dma = pltpu.make_async_copy(
                out_stage.at[oslot],
                out_hbm.at[pl.ds(row_start, M_LOC), pl.ds(col_block * CB, CB)],
                out_sems.at[oslot],
            )
            dma.start()
            out_inflight[oslot] = dma
            oslot = (oslot + 1) % 2

        for idx, (s, j) in enumerate(order):
            t = (my + s) % N_DEV
            if idx + 1 < len(order):
                s2, j2 = order[idx + 1]
                t2 = (my + s2) % N_DEV
                loads[idx + 1] = w_load(t2, j2, (idx + 1) % 2)
                loads[idx + 1].start()
            loads[idx].wait()

            y = jnp.dot(x_ref[...], w_buf[idx % 2],
                        preferred_element_type=jnp.float32)
            g = _gelu(y)

            if s != 0:
                sb = (s - 1) * NBLK + j
                send_buf[sb] = g.astype(jnp.bfloat16)
                rslot = (3 - s) * NBLK + j
                rdma = pltpu.make_async_remote_copy(
                    src_ref=send_buf.at[sb],
                    dst_ref=recv_buf.at[rslot],
                    send_sem=send_sems.at[sb],
                    recv_sem=recv_sems.at[rslot],
                    device_id=(t,),
                    device_id_type=pl.DeviceIdType.MESH,
                )
                rdma.start()
                send_rdmas.append(rdma)
            else:
                store_out(g, my * M_LOC, j)

        for s in (3, 2, 1):
            o = (my + s) % N_DEV
            for j in range(NBLK):
                rslot = (s - 1) * NBLK + j
                recv = pltpu.make_async_remote_copy(
                    src_ref=send_buf.at[0],
                    dst_ref=recv_buf.at[rslot],
                    send_sem=send_sems.at[0],
                    recv_sem=recv_sems.at[rslot],
                    device_id=(my,),
                    device_id_type=pl.DeviceIdType.MESH,
                )
                recv.wait_recv()
                store_out(recv_buf[rslot].astype(jnp.float32), o * M_LOC, j)

        for rdma in send_rdmas:
            rdma.wait_send()
        for dma in out_inflight:
            if dma is not None:
                dma.wait()

    out_shape = jax.ShapeDtypeStruct((N_DEV * M_LOC, N_LOC), jnp.float32)
    return pl.pallas_call(
        body,
        out_shape=out_shape,
        in_specs=[
            pl.BlockSpec(memory_space=pltpu.MemorySpace.VMEM),
            pl.BlockSpec(memory_space=pltpu.MemorySpace.HBM),
        ],
        out_specs=pl.BlockSpec(memory_space=pltpu.MemorySpace.HBM),
        scratch_shapes=[
            pltpu.VMEM((2, K, CB), jnp.bfloat16),
            pltpu.VMEM(((N_DEV - 1) * NBLK, M_LOC, CB), jnp.bfloat16),
            pltpu.VMEM(((N_DEV - 1) * NBLK, M_LOC, CB), jnp.bfloat16),
            pltpu.VMEM((2, M_LOC, CB), jnp.float32),
            pltpu.SemaphoreType.DMA((2,)),
            pltpu.SemaphoreType.DMA(((N_DEV - 1) * NBLK,)),
            pltpu.SemaphoreType.DMA(((N_DEV - 1) * NBLK,)),
            pltpu.SemaphoreType.DMA((2,)),
        ],
        compiler_params=pltpu.CompilerParams(collective_id=0),
    )(x, w_mat)


# baseline (device time: 174644 ns/iter reference)
import jax
import jax.numpy as jnp
from jax import lax
from jax.experimental import pallas as pl
from jax.experimental.pallas import tpu as pltpu

N_DEV = 4
M_LOC = 1024
K = 4096
KB = K // 2
N_GLOB = 8192
N_LOC = N_GLOB // N_DEV
CB = 512
NBLK = N_LOC // CB

GELU_C = 0.7978845608028654


def _gelu(y):
    return 0.5 * y * (1.0 + jnp.tanh(GELU_C * (y + 0.044715 * y * y * y)))


def kernel(x, w_mat):
    def body(x_ref, w_hbm, out_hbm, w_buf, send_buf, recv_buf, out_stage,
             w_sems, send_sems, recv_sems, out_sems):
        my = lax.axis_index("i")

        barrier = pltpu.get_barrier_semaphore()
        for s in range(1, N_DEV):
            pl.semaphore_signal(
                barrier, inc=1,
                device_id=((my + s) % N_DEV,),
                device_id_type=pl.DeviceIdType.MESH,
            )
        pl.semaphore_wait(barrier, N_DEV - 1)

        order = [(s, j) for s in (1, 2, 3, 0) for j in range(NBLK)]
        steps = [(s, j, kb) for (s, j) in order for kb in range(2)]

        def w_load(step_idx):
            s, j, kb = steps[step_idx]
            t = (my + s) % N_DEV
            cb = t * NBLK + j
            slot = step_idx % 2
            return pltpu.make_async_copy(
                w_hbm.at[pl.ds(kb * KB, KB), pl.ds(cb * CB, CB)],
                w_buf.at[slot],
                w_sems.at[slot],
            )

        loads = {0: w_load(0)}
        loads[0].start()

        out_inflight = [None, None]
        send_rdmas = []
        oslot = 0

        def store_out(value_f32, row_start, col_start):
            nonlocal oslot
            if out_inflight[oslot] is not None:
                out_inflight[oslot].wait()
            out_stage[oslot] = value_f32
            dma = pltpu.make_async_copy(
                out_stage.at[oslot],
                out_hbm.at[pl.ds(row_start, M_LOC), pl.ds(col_start, CB)],
                out_sems.at[oslot],
            )
            dma.start()
            out_inflight[oslot] = dma
            oslot = (oslot + 1) % 2

        y_partial = None
        for step_idx, (s, j, kb) in enumerate(steps):
            t = (my + s) % N_DEV
            if step_idx + 1 < len(steps):
                loads[step_idx + 1] = w_load(step_idx + 1)
                loads[step_idx + 1].start()
            loads[step_idx].wait()

            wblk = w_buf[step_idx % 2].astype(jnp.bfloat16)
            yk = jnp.dot(x_ref[:, kb * KB:(kb + 1) * KB], wblk,
                         preferred_element_type=jnp.float32)
            if kb == 0:
                y_partial = yk
                continue
            g = _gelu(y_partial + yk)

            if s != 0:
                sb = (s - 1) * NBLK + j
                send_buf[sb] = g.astype(jnp.bfloat16)
                rslot = (3 - s) * NBLK + j
                rdma = pltpu.make_async_remote_copy(
                    src_ref=send_buf.at[sb],
                    dst_ref=recv_buf.at[rslot],
                    send_sem=send_sems.at[sb],
                    recv_sem=recv_sems.at[rslot],
                    device_id=(t,),
                    device_id_type=pl.DeviceIdType.MESH,
                )
                rdma.start()
                send_rdmas.append(rdma)
            else:
                store_out(g, my * M_LOC, j * CB)

        for s in (3, 2, 1):
            o = (my + s) % N_DEV
            for j in range(NBLK):
                rslot = (s - 1) * NBLK + j
                recv = pltpu.make_async_remote_copy(
                    src_ref=send_buf.at[0],
                    dst_ref=recv_buf.at[rslot],
                    send_sem=send_sems.at[0],
                    recv_sem=recv_sems.at[rslot],
                    device_id=(my,),
                    device_id_type=pl.DeviceIdType.MESH,
                )
                recv.wait_recv()
                store_out(recv_buf[rslot].astype(jnp.float32),
                          o * M_LOC, j * CB)

        for rdma in send_rdmas:
            rdma.wait_send()
        for dma in out_inflight:
            if dma is not None:
                dma.wait()

    out_shape = jax.ShapeDtypeStruct((N_DEV * M_LOC, N_LOC), jnp.float32)
    n_msg = (N_DEV - 1) * NBLK
    return pl.pallas_call(
        body,
        out_shape=out_shape,
        in_specs=[
            pl.BlockSpec(memory_space=pltpu.MemorySpace.VMEM),
            pl.BlockSpec(memory_space=pltpu.MemorySpace.HBM),
        ],
        out_specs=pl.BlockSpec(memory_space=pltpu.MemorySpace.HBM),
        scratch_shapes=[
            pltpu.VMEM((2, KB, CB), jnp.float32),
            pltpu.VMEM((n_msg, M_LOC, CB), jnp.bfloat16),
            pltpu.VMEM((n_msg, M_LOC, CB), jnp.bfloat16),
            pltpu.VMEM((2, M_LOC, CB), jnp.float32),
            pltpu.SemaphoreType.DMA((2,)),
            pltpu.SemaphoreType.DMA((n_msg,)),
            pltpu.SemaphoreType.DMA((n_msg,)),
            pltpu.SemaphoreType.DMA((2,)),
        ],
        compiler_params=pltpu.CompilerParams(
            collective_id=0,
            vmem_limit_bytes=56 * 1024 * 1024,
        ),
    )(x.astype(jnp.bfloat16), w_mat)


# device time: 147160 ns/iter; 1.1868x vs baseline; 1.1868x over previous
import jax
import jax.numpy as jnp
from jax import lax
from jax.experimental import pallas as pl
from jax.experimental.pallas import tpu as pltpu

N_DEV = 4
M_LOC = 1024
K = 4096
KB = K // 2
N_GLOB = 8192
N_LOC = N_GLOB // N_DEV
CB = 512
NBLK = N_LOC // CB
W_SLOTS = 3

GELU_C = 0.7978845608028654


def _gelu(y):
    return 0.5 * y * (1.0 + jnp.tanh(GELU_C * (y + 0.044715 * y * y * y)))


def kernel(x, w_mat):
    def body(x_ref, w_hbm, out_hbm, w_buf, send_buf, recv_buf, out_stage,
             w_sems, send_sems, recv_sems, out_sems):
        my = lax.axis_index("i")

        barrier = pltpu.get_barrier_semaphore()
        for s in range(1, N_DEV):
            pl.semaphore_signal(
                barrier, inc=1,
                device_id=((my + s) % N_DEV,),
                device_id_type=pl.DeviceIdType.MESH,
            )
        pl.semaphore_wait(barrier, N_DEV - 1)

        order = [(s, j) for j in range(NBLK) for s in (1, 2, 3)]
        order += [(0, j) for j in range(NBLK)]
        steps = [(s, j, kb) for (s, j) in order for kb in range(2)]

        def w_load(step_idx):
            s, j, kb = steps[step_idx]
            t = (my + s) % N_DEV
            cb = t * NBLK + j
            slot = step_idx % W_SLOTS
            return pltpu.make_async_copy(
                w_hbm.at[pl.ds(kb * KB, KB), pl.ds(cb * CB, CB)],
                w_buf.at[slot],
                w_sems.at[slot],
            )

        loads = {}
        for i in range(W_SLOTS - 1):
            loads[i] = w_load(i)
            loads[i].start()

        out_inflight = [None, None]
        send_rdmas = []
        oslot = 0

        def store_out(value_f32, row_start, col_start):
            nonlocal oslot
            if out_inflight[oslot] is not None:
                out_inflight[oslot].wait()
            out_stage[oslot] = value_f32
            dma = pltpu.make_async_copy(
                out_stage.at[oslot],
                out_hbm.at[pl.ds(row_start, M_LOC), pl.ds(col_start, CB)],
                out_sems.at[oslot],
            )
            dma.start()
            out_inflight[oslot] = dma
            oslot = (oslot + 1) % 2

        y_partial = None
        for step_idx, (s, j, kb) in enumerate(steps):
            t = (my + s) % N_DEV
            nxt = step_idx + W_SLOTS - 1
            if nxt < len(steps):
                loads[nxt] = w_load(nxt)
                loads[nxt].start()
            loads[step_idx].wait()

            wblk = w_buf[step_idx % W_SLOTS].astype(jnp.bfloat16)
            yk = jnp.dot(x_ref[:, kb * KB:(kb + 1) * KB], wblk,
                         preferred_element_type=jnp.float32)
            if kb == 0:
                y_partial = yk
                continue
            g = _gelu(y_partial + yk)

            if s != 0:
                sb = (s - 1) * NBLK + j
                send_buf[sb] = g.astype(jnp.bfloat16)
                rslot = (3 - s) * NBLK + j
                rdma = pltpu.make_async_remote_copy(
                    src_ref=send_buf.at[sb],
                    dst_ref=recv_buf.at[rslot],
                    send_sem=send_sems.at[sb],
                    recv_sem=recv_sems.at[rslot],
                    device_id=(t,),
                    device_id_type=pl.DeviceIdType.MESH,
                )
                rdma.start()
                send_rdmas.append(rdma)
            else:
                store_out(g, my * M_LOC, j * CB)

        for s in (3, 2, 1):
            o = (my + s) % N_DEV
            for j in range(NBLK):
                rslot = (s - 1) * NBLK + j
                recv = pltpu.make_async_remote_copy(
                    src_ref=send_buf.at[0],
                    dst_ref=recv_buf.at[rslot],
                    send_sem=send_sems.at[0],
                    recv_sem=recv_sems.at[rslot],
                    device_id=(my,),
                    device_id_type=pl.DeviceIdType.MESH,
                )
                recv.wait_recv()
                store_out(recv_buf[rslot].astype(jnp.float32),
                          o * M_LOC, j * CB)

        for rdma in send_rdmas:
            rdma.wait_send()
        for dma in out_inflight:
            if dma is not None:
                dma.wait()

    out_shape = jax.ShapeDtypeStruct((N_DEV * M_LOC, N_LOC), jnp.float32)
    n_msg = (N_DEV - 1) * NBLK
    return pl.pallas_call(
        body,
        out_shape=out_shape,
        in_specs=[
            pl.BlockSpec(memory_space=pltpu.MemorySpace.VMEM),
            pl.BlockSpec(memory_space=pltpu.MemorySpace.HBM),
        ],
        out_specs=pl.BlockSpec(memory_space=pltpu.MemorySpace.HBM),
        scratch_shapes=[
            pltpu.VMEM((W_SLOTS, KB, CB), jnp.float32),
            pltpu.VMEM((n_msg, M_LOC, CB), jnp.bfloat16),
            pltpu.VMEM((n_msg, M_LOC, CB), jnp.bfloat16),
            pltpu.VMEM((2, M_LOC, CB), jnp.float32),
            pltpu.SemaphoreType.DMA((W_SLOTS,)),
            pltpu.SemaphoreType.DMA((n_msg,)),
            pltpu.SemaphoreType.DMA((n_msg,)),
            pltpu.SemaphoreType.DMA((2,)),
        ],
        compiler_params=pltpu.CompilerParams(
            collective_id=0,
            vmem_limit_bytes=56 * 1024 * 1024,
        ),
    )(x.astype(jnp.bfloat16), w_mat)


# device time: 134280 ns/iter; 1.3006x vs baseline; 1.0959x over previous
import os

import jax
import jax.numpy as jnp
from jax import lax
from jax.experimental import pallas as pl
from jax.experimental.pallas import tpu as pltpu

_VARIANT = os.environ.get("KERNEL_VARIANT", "full")

N_DEV = 4
M_LOC = 1024
K = 4096
KB = K // 2
N_GLOB = 8192
N_LOC = N_GLOB // N_DEV
CB = 512
NBLK = N_LOC // CB
W_SLOTS = 3

GELU_C = 0.7978845608028654


def _gelu(y):
    return 0.5 * y * (1.0 + jnp.tanh(GELU_C * (y + 0.044715 * y * y * y)))


def kernel(x, w_mat):
    def body(x_ref, w_hbm, out_hbm, w_buf, send_buf, recv_buf, out_stage,
             w_sems, send_sems, recv_sems, out_sems):
        my = lax.axis_index("i")

        barrier = pltpu.get_barrier_semaphore()
        for s in range(1, N_DEV):
            pl.semaphore_signal(
                barrier, inc=1,
                device_id=((my + s) % N_DEV,),
                device_id_type=pl.DeviceIdType.MESH,
            )
        pl.semaphore_wait(barrier, N_DEV - 1)

        order = [(s, j) for j in range(NBLK) for s in (1, 2, 3)]
        order += [(0, j) for j in range(NBLK)]
        steps = [(s, j, kb) for (s, j) in order for kb in range(2)]

        def w_load(step_idx):
            s, j, kb = steps[step_idx]
            t = (my + s) % N_DEV
            cb = t * NBLK + j
            slot = step_idx % W_SLOTS
            return pltpu.make_async_copy(
                w_hbm.at[pl.ds(kb * KB, KB), pl.ds(cb * CB, CB)],
                w_buf.at[slot],
                w_sems.at[slot],
            )

        loads = {}
        if _VARIANT != "comm":
            for i in range(W_SLOTS - 1):
                loads[i] = w_load(i)
                loads[i].start()

        out_inflight = [None, None]
        send_rdmas = []
        oslot = 0

        def store_out(value_f32, row_start, col_start):
            nonlocal oslot
            if out_inflight[oslot] is not None:
                out_inflight[oslot].wait()
            out_stage[oslot] = value_f32
            dma = pltpu.make_async_copy(
                out_stage.at[oslot],
                out_hbm.at[pl.ds(row_start, M_LOC), pl.ds(col_start, CB)],
                out_sems.at[oslot],
            )
            dma.start()
            out_inflight[oslot] = dma
            oslot = (oslot + 1) % 2

        y_partial = None
        for step_idx, (s, j, kb) in enumerate(steps):
            t = (my + s) % N_DEV
            if _VARIANT == "comm":
                if kb == 0:
                    continue
                g = None
            else:
                nxt = step_idx + W_SLOTS - 1
                if nxt < len(steps):
                    loads[nxt] = w_load(nxt)
                    loads[nxt].start()
                loads[step_idx].wait()

                wblk = w_buf[step_idx % W_SLOTS].astype(jnp.bfloat16)
                yk = jnp.dot(x_ref[:, kb * KB:(kb + 1) * KB], wblk,
                             preferred_element_type=jnp.float32)
                if kb == 0:
                    y_partial = yk
                    continue
                g = _gelu(y_partial + yk)

            if _VARIANT == "compute":
                store_out(g, my * M_LOC, j * CB)
                continue

            if s != 0:
                sb = (s - 1) * NBLK + j
                if _VARIANT == "comm":
                    send_buf[sb] = x_ref[:, :CB]
                else:
                    send_buf[sb] = g.astype(jnp.bfloat16)
                rslot = (3 - s) * NBLK + j
                rdma = pltpu.make_async_remote_copy(
                    src_ref=send_buf.at[sb],
                    dst_ref=recv_buf.at[rslot],
                    send_sem=send_sems.at[sb],
                    recv_sem=recv_sems.at[rslot],
                    device_id=(t,),
                    device_id_type=pl.DeviceIdType.MESH,
                )
                rdma.start()
                send_rdmas.append(rdma)
            else:
                if _VARIANT != "comm":
                    store_out(g, my * M_LOC, j * CB)

        for s in (3, 2, 1) if _VARIANT != "compute" else ():
            o = (my + s) % N_DEV
            for j in range(NBLK):
                rslot = (s - 1) * NBLK + j
                recv = pltpu.make_async_remote_copy(
                    src_ref=send_buf.at[0],
                    dst_ref=recv_buf.at[rslot],
                    send_sem=send_sems.at[0],
                    recv_sem=recv_sems.at[rslot],
                    device_id=(my,),
                    device_id_type=pl.DeviceIdType.MESH,
                )
                recv.wait_recv()
                store_out(recv_buf[rslot].astype(jnp.float32),
                          o * M_LOC, j * CB)

        for rdma in send_rdmas:
            rdma.wait_send()
        for dma in out_inflight:
            if dma is not None:
                dma.wait()

    out_shape = jax.ShapeDtypeStruct((N_DEV * M_LOC, N_LOC), jnp.float32)
    n_msg = (N_DEV - 1) * NBLK
    return pl.pallas_call(
        body,
        out_shape=out_shape,
        in_specs=[
            pl.BlockSpec(memory_space=pltpu.MemorySpace.VMEM),
            pl.BlockSpec(memory_space=pltpu.MemorySpace.HBM),
        ],
        out_specs=pl.BlockSpec(memory_space=pltpu.MemorySpace.HBM),
        scratch_shapes=[
            pltpu.VMEM((W_SLOTS, KB, CB), jnp.float32),
            pltpu.VMEM((n_msg, M_LOC, CB), jnp.bfloat16),
            pltpu.VMEM((n_msg, M_LOC, CB), jnp.bfloat16),
            pltpu.VMEM((2, M_LOC, CB), jnp.float32),
            pltpu.SemaphoreType.DMA((W_SLOTS,)),
            pltpu.SemaphoreType.DMA((n_msg,)),
            pltpu.SemaphoreType.DMA((n_msg,)),
            pltpu.SemaphoreType.DMA((2,)),
        ],
        compiler_params=pltpu.CompilerParams(
            collective_id=0,
            vmem_limit_bytes=56 * 1024 * 1024,
        ),
    )(x.astype(jnp.bfloat16), w_mat)
